# Initial kernel scaffold; baseline (speedup 1.0000x reference)
import functools

import jax
import jax.numpy as jnp
from jax import lax
from jax.experimental import pallas as pl
from jax.experimental.pallas import tpu as pltpu
from jax.experimental.pallas import tpu_sc as plsc

NN = 10000
NG = 64
N_PAD = 10240
NC, NS = 2, 16
NW = NC * NS
K = 128
E_PAD = 327680
CH_PER_TILE = E_PAD // (NW * K)
ROWS_PER_TILE = N_PAD // NS

_f32 = jnp.float32


def _tc_embed_body(hin_ref, w_ref, asrc_ref, adst_ref,
                   h_ref, astab_ref, adtab_ref, cvec_ref):
    h = jnp.dot(hin_ref[...], w_ref[...], preferred_element_type=_f32)
    h_ref[...] = h
    a_s = jnp.sum(h * asrc_ref[...], axis=1, keepdims=True)
    a_d = jnp.sum(h * adst_ref[...], axis=1, keepdims=True)
    astab_ref[...] = a_s
    adtab_ref[...] = a_d
    c = jnp.maximum(jnp.max(a_s) + jnp.max(a_d), 0.0)
    cvec_ref[...] = jnp.full((1, 16), c, _f32)


def _tc_embed(hin, w, asrc, adst):
    dout = w.shape[1]
    return pl.pallas_call(
        _tc_embed_body,
        out_shape=[
            jax.ShapeDtypeStruct((N_PAD, dout), _f32),
            jax.ShapeDtypeStruct((N_PAD, 1), _f32),
            jax.ShapeDtypeStruct((N_PAD, 1), _f32),
            jax.ShapeDtypeStruct((1, 16), _f32),
        ],
    )(hin, w, asrc.reshape(1, -1), adst.reshape(1, -1))


def _tc_mid_body(accf_ref, accd_ref, b_ref, w_ref, asrc_ref, adst_ref,
                 h_ref, astab_ref, adtab_ref, cvec_ref):
    feat = accf_ref[0] + accf_ref[1]
    den = accd_ref[0, :, 0:1] + accd_ref[1, :, 0:1]
    hprev = feat / (den + 1e-16) + b_ref[...]
    hprev = jnp.maximum(hprev, 0.0)
    h = jnp.dot(hprev, w_ref[...], preferred_element_type=_f32)
    h_ref[...] = h
    a_s = jnp.sum(h * asrc_ref[...], axis=1, keepdims=True)
    a_d = jnp.sum(h * adst_ref[...], axis=1, keepdims=True)
    astab_ref[...] = a_s
    adtab_ref[...] = a_d
    c = jnp.maximum(jnp.max(a_s) + jnp.max(a_d), 0.0)
    cvec_ref[...] = jnp.full((1, 16), c, _f32)


def _tc_mid(accf, accd, b, w, asrc, adst):
    dout = w.shape[1]
    return pl.pallas_call(
        _tc_mid_body,
        out_shape=[
            jax.ShapeDtypeStruct((N_PAD, dout), _f32),
            jax.ShapeDtypeStruct((N_PAD, 1), _f32),
            jax.ShapeDtypeStruct((N_PAD, 1), _f32),
            jax.ShapeDtypeStruct((1, 16), _f32),
        ],
    )(accf, accd, b.reshape(1, -1), w, asrc.reshape(1, -1), adst.reshape(1, -1))


def _tc_final_body(accf_ref, accd_ref, b_ref, batch_ref,
                   l1w_ref, l1b_ref, l2w_ref, l2b_ref, out_ref):
    feat = accf_ref[0] + accf_ref[1]
    den = accd_ref[0, :, 0:1] + accd_ref[1, :, 0:1]
    h = feat / (den + 1e-16) + b_ref[...]
    batch = batch_ref[...]
    gid = lax.broadcasted_iota(jnp.int32, (NG, N_PAD), 0)
    oh = (batch == gid).astype(_f32)
    g = jnp.dot(oh, h, preferred_element_type=_f32)
    g = jnp.maximum(jnp.dot(g, l1w_ref[...], preferred_element_type=_f32)
                    + l1b_ref[...], 0.0)
    z = jnp.dot(g, l2w_ref[...], preferred_element_type=_f32) + l2b_ref[...]
    m0 = jnp.max(z, axis=0, keepdims=True)
    z = z - m0
    out_ref[...] = z - jnp.log(jnp.sum(jnp.exp(z), axis=0, keepdims=True))


def _tc_final(accf, accd, b, batch_p, l1w, l1b, l2w, l2b):
    nclass = l2w.shape[1]
    return pl.pallas_call(
        _tc_final_body,
        out_shape=jax.ShapeDtypeStruct((NG, nclass), _f32),
    )(accf, accd, b.reshape(1, -1), batch_p,
      l1w, l1b.reshape(1, -1), l2w, l2b.reshape(1, -1))


@functools.cache
def _make_sc_edge(d):
    d16 = d // 16
    mesh = plsc.VectorSubcoreMesh(core_axis_name="c", subcore_axis_name="s")

    @functools.partial(
        pl.kernel,
        out_type=(
            jax.ShapeDtypeStruct((NC, N_PAD, d), _f32),
            jax.ShapeDtypeStruct((NC, N_PAD, 16), _f32),
        ),
        mesh=mesh,
        scratch_types=[
            pltpu.VMEM((N_PAD,), _f32),
            pltpu.VMEM((N_PAD,), _f32),
            pltpu.VMEM((16,), _f32),
            pltpu.VMEM((K,), jnp.int32),
            pltpu.VMEM((K,), jnp.int32),
            pltpu.VMEM((K,), _f32),
            pltpu.VMEM((K, 128), _f32),
            pltpu.VMEM((K, 16), _f32),
            pltpu.VMEM_SHARED((N_PAD, 128), _f32),
            pltpu.VMEM_SHARED((N_PAD, 16), _f32),
            pltpu.SemaphoreType.DMA,
        ],
    )
    def sc_edge(h_hbm, astab_hbm, adtab_hbm, cvec_hbm, src_hbm, dst_hbm,
                accf_hbm, accd_hbm,
                as_v, ad_v, cv_v, sidx_v, didx_v, p_v, rows_full_v, den_v,
                accf_full_s, accd_s, sem):
        rows_v = rows_full_v.at[:, :d]
        accf_s = accf_full_s.at[:, :d]
        cid = lax.axis_index("c")
        sid = lax.axis_index("s")
        wid = sid * NC + cid

        def zrow(k, _):
            for j in range(d16):
                rows_v[k, pl.ds(j * 16, 16)] = jnp.zeros((16,), _f32)
            den_v[k, :] = jnp.zeros((16,), _f32)
            return 0
        lax.fori_loop(0, K, zrow, 0)
        for i in range(ROWS_PER_TILE // K):
            r0 = sid * ROWS_PER_TILE + i * K
            pltpu.sync_copy(rows_v, accf_s.at[pl.ds(r0, K)])
            pltpu.sync_copy(den_v, accd_s.at[pl.ds(r0, K)])

        pltpu.sync_copy(astab_hbm, as_v)
        pltpu.sync_copy(adtab_hbm, ad_v)
        pltpu.sync_copy(cvec_hbm, cv_v)
        plsc.subcore_barrier()

        def chunk(t, _):
            ch = wid * CH_PER_TILE + t
            pltpu.sync_copy(src_hbm.at[ch], sidx_v)
            pltpu.sync_copy(dst_hbm.at[ch], didx_v)
            pltpu.async_copy(h_hbm.at[sidx_v], rows_v, sem).wait()
            cv = cv_v[:]
            for g in range(K // 16):
                s16 = sidx_v[pl.ds(g * 16, 16)]
                t16 = didx_v[pl.ds(g * 16, 16)]
                av = plsc.load_gather(as_v, [s16])
                bv = plsc.load_gather(ad_v, [t16])
                e = av + bv
                e = jnp.where(e >= 0.0, e, e * 0.2)
                p_v[pl.ds(g * 16, 16)] = jnp.exp(e - cv)

            def scale(k, _):
                pk = plsc.load_gather(p_v, [jnp.full((16,), k, jnp.int32)])
                for j in range(d16):
                    rows_v[k, pl.ds(j * 16, 16)] = (
                        rows_v[k, pl.ds(j * 16, 16)] * pk)
                den_v[k, :] = pk
                return 0
            lax.fori_loop(0, K, scale, 0)

            pltpu.sync_copy(rows_v, accf_s.at[didx_v], add=True)
            pltpu.sync_copy(den_v, accd_s.at[didx_v], add=True)
            return 0
        lax.fori_loop(0, CH_PER_TILE, chunk, 0)
        plsc.subcore_barrier()

        for i in range(ROWS_PER_TILE // K):
            rsl = pl.ds(sid * ROWS_PER_TILE + i * K, K)
            pltpu.sync_copy(accf_s.at[rsl], accf_hbm.at[cid, rsl])
            pltpu.sync_copy(accd_s.at[rsl], accd_hbm.at[cid, rsl])

    return sc_edge


def kernel(x, edge_index, batch, W1, a1_src, a1_dst, b1, W2, a2_src, a2_dst,
           b2, W3, a3_src, a3_dst, b3, L1W, L1b, L2W, L2b):
    n, e = x.shape[0], edge_index.shape[1]
    x_pad = jnp.zeros((N_PAD, x.shape[1]), _f32).at[:n].set(x)
    pad_e = E_PAD - e
    src_p = jnp.concatenate(
        [edge_index[0], jnp.full((pad_e,), NN, jnp.int32)]).reshape(-1, K)
    dst_p = jnp.concatenate(
        [edge_index[1], jnp.full((pad_e,), NN, jnp.int32)]).reshape(-1, K)
    batch_p = jnp.concatenate(
        [batch, jnp.full((N_PAD - n,), NG, jnp.int32)]).reshape(1, N_PAD)

    h, astab, adtab, cvec = _tc_embed(x_pad, W1, a1_src, a1_dst)
    accf, accd = _make_sc_edge(W1.shape[1])(
        h, astab.reshape(-1), adtab.reshape(-1), cvec.reshape(-1),
        src_p, dst_p)
    h, astab, adtab, cvec = _tc_mid(accf, accd, b1, W2, a2_src, a2_dst)
    accf, accd = _make_sc_edge(W2.shape[1])(
        h, astab.reshape(-1), adtab.reshape(-1), cvec.reshape(-1),
        src_p, dst_p)
    h, astab, adtab, cvec = _tc_mid(accf, accd, b2, W3, a3_src, a3_dst)
    accf, accd = _make_sc_edge(W3.shape[1])(
        h, astab.reshape(-1), adtab.reshape(-1), cvec.reshape(-1),
        src_p, dst_p)
    return _tc_final(accf, accd, b3, batch_p, L1W, L1b, L2W, L2b)


# speedup vs baseline: 18.4376x; 18.4376x over previous
import functools

import jax
import jax.numpy as jnp
from jax import lax
from jax.experimental import pallas as pl
from jax.experimental.pallas import tpu as pltpu
from jax.experimental.pallas import tpu_sc as plsc

NN = 10000
NG = 64
N_PAD = 10176
NC, NS = 2, 16
NW = NC * NS
K = 128
E_PAD = 327680
CH_PER_TILE = E_PAD // (NW * K)
ROWS_PER_TILE = N_PAD // NS

_f32 = jnp.float32


def _tc_embed_body(hin_ref, w_ref, asrc_ref, adst_ref,
                   h_ref, astab_ref, adtab_ref, cvec_ref):
    h = jnp.dot(hin_ref[...], w_ref[...], preferred_element_type=_f32)
    h_ref[...] = h
    a_s = jnp.sum(h * asrc_ref[...], axis=1, keepdims=True)
    a_d = jnp.sum(h * adst_ref[...], axis=1, keepdims=True)
    astab_ref[...] = a_s
    adtab_ref[...] = a_d
    c = jnp.maximum(jnp.max(a_s) + jnp.max(a_d), 0.0)
    cvec_ref[...] = jnp.full((1, 16), c, _f32)


def _tc_embed(hin, w, asrc, adst):
    dout = w.shape[1]
    return pl.pallas_call(
        _tc_embed_body,
        out_shape=[
            jax.ShapeDtypeStruct((N_PAD, dout), _f32),
            jax.ShapeDtypeStruct((N_PAD, 1), _f32),
            jax.ShapeDtypeStruct((N_PAD, 1), _f32),
            jax.ShapeDtypeStruct((1, 16), _f32),
        ],
    )(hin, w, asrc.reshape(1, -1), adst.reshape(1, -1))


def _tc_mid_body(accf_ref, accd_ref, b_ref, w_ref, asrc_ref, adst_ref,
                 h_ref, astab_ref, adtab_ref, cvec_ref):
    feat = accf_ref[0] + accf_ref[1]
    den = accd_ref[0, :, 0:1] + accd_ref[1, :, 0:1]
    hprev = feat / (den + 1e-16) + b_ref[...]
    hprev = jnp.maximum(hprev, 0.0)
    h = jnp.dot(hprev, w_ref[...], preferred_element_type=_f32)
    h_ref[...] = h
    a_s = jnp.sum(h * asrc_ref[...], axis=1, keepdims=True)
    a_d = jnp.sum(h * adst_ref[...], axis=1, keepdims=True)
    astab_ref[...] = a_s
    adtab_ref[...] = a_d
    c = jnp.maximum(jnp.max(a_s) + jnp.max(a_d), 0.0)
    cvec_ref[...] = jnp.full((1, 16), c, _f32)


def _tc_mid(accf, accd, b, w, asrc, adst):
    dout = w.shape[1]
    return pl.pallas_call(
        _tc_mid_body,
        out_shape=[
            jax.ShapeDtypeStruct((N_PAD, dout), _f32),
            jax.ShapeDtypeStruct((N_PAD, 1), _f32),
            jax.ShapeDtypeStruct((N_PAD, 1), _f32),
            jax.ShapeDtypeStruct((1, 16), _f32),
        ],
    )(accf, accd, b.reshape(1, -1), w, asrc.reshape(1, -1), adst.reshape(1, -1))


def _tc_final_body(accf_ref, accd_ref, b_ref, batch_ref,
                   l1w_ref, l1b_ref, l2w_ref, l2b_ref, out_ref):
    feat = accf_ref[0] + accf_ref[1]
    den = accd_ref[0, :, 0:1] + accd_ref[1, :, 0:1]
    h = feat / (den + 1e-16) + b_ref[...]
    batch = batch_ref[...]
    gid = lax.broadcasted_iota(jnp.int32, (NG, N_PAD), 0)
    oh = (batch == gid).astype(_f32)
    g = jnp.dot(oh, h, preferred_element_type=_f32)
    g = jnp.maximum(jnp.dot(g, l1w_ref[...], preferred_element_type=_f32)
                    + l1b_ref[...], 0.0)
    z = jnp.dot(g, l2w_ref[...], preferred_element_type=_f32) + l2b_ref[...]
    m0 = jnp.max(z, axis=0, keepdims=True)
    z = z - m0
    out_ref[...] = z - jnp.log(jnp.sum(jnp.exp(z), axis=0, keepdims=True))


def _tc_final(accf, accd, b, batch_p, l1w, l1b, l2w, l2b):
    nclass = l2w.shape[1]
    return pl.pallas_call(
        _tc_final_body,
        out_shape=jax.ShapeDtypeStruct((NG, nclass), _f32),
    )(accf, accd, b.reshape(1, -1), batch_p,
      l1w, l1b.reshape(1, -1), l2w, l2b.reshape(1, -1))


@functools.cache
def _make_sc_edge(d):
    d16 = d // 16
    mesh = plsc.VectorSubcoreMesh(core_axis_name="c", subcore_axis_name="s")

    @functools.partial(
        pl.kernel,
        out_type=(
            jax.ShapeDtypeStruct((NC, N_PAD, d), _f32),
            jax.ShapeDtypeStruct((NC, N_PAD, 16), _f32),
        ),
        mesh=mesh,
        compiler_params=pltpu.CompilerParams(
            needs_layout_passes=False, use_tc_tiling_on_sc=False),
        scratch_types=[
            pltpu.VMEM((N_PAD,), _f32),
            pltpu.VMEM((N_PAD,), _f32),
            pltpu.VMEM((16,), _f32),
            pltpu.VMEM((K,), jnp.int32),
            pltpu.VMEM((K,), jnp.int32),
            pltpu.VMEM((K,), _f32),
            pltpu.VMEM((K, d), _f32),
            pltpu.VMEM((K, 16), _f32),
            pltpu.VMEM_SHARED((N_PAD, d), _f32),
            pltpu.VMEM_SHARED((N_PAD, 16), _f32),
            pltpu.SemaphoreType.DMA,
        ],
    )
    def sc_edge(h_hbm, astab_hbm, adtab_hbm, cvec_hbm, src_hbm, dst_hbm,
                accf_hbm, accd_hbm,
                as_v, ad_v, cv_v, sidx_v, didx_v, p_v, rows_v, den_v,
                accf_s, accd_s, sem):
        cid = lax.axis_index("c")
        sid = lax.axis_index("s")
        wid = sid * NC + cid

        def zrow(k, _):
            for j in range(d16):
                rows_v[k, pl.ds(j * 16, 16)] = jnp.zeros((16,), _f32)
            den_v[k, :] = jnp.zeros((16,), _f32)
            return 0
        lax.fori_loop(0, K, zrow, 0)
        row_chunks = [(o, min(K, ROWS_PER_TILE - o))
                      for o in range(0, ROWS_PER_TILE, K)]
        for o, cnt in row_chunks:
            r0 = sid * ROWS_PER_TILE + o
            pltpu.sync_copy(rows_v.at[pl.ds(0, cnt)], accf_s.at[pl.ds(r0, cnt)])
            pltpu.sync_copy(den_v.at[pl.ds(0, cnt)], accd_s.at[pl.ds(r0, cnt)])

        pltpu.sync_copy(astab_hbm, as_v)
        pltpu.sync_copy(adtab_hbm, ad_v)
        pltpu.sync_copy(cvec_hbm, cv_v)
        plsc.subcore_barrier()

        def chunk(t, _):
            ch = wid * CH_PER_TILE + t
            pltpu.sync_copy(src_hbm.at[ch], sidx_v)
            pltpu.sync_copy(dst_hbm.at[ch], didx_v)
            pltpu.async_copy(h_hbm.at[sidx_v], rows_v, sem).wait()
            cv = cv_v[:]
            for g in range(K // 16):
                s16 = sidx_v[pl.ds(g * 16, 16)]
                t16 = didx_v[pl.ds(g * 16, 16)]
                av = plsc.load_gather(as_v, [s16])
                bv = plsc.load_gather(ad_v, [t16])
                e = av + bv
                e = jnp.where(e >= 0.0, e, e * 0.2)
                p_v[pl.ds(g * 16, 16)] = jnp.exp(e - cv)

            def scale(k, _):
                pk = plsc.load_gather(p_v, [jnp.full((16,), k, jnp.int32)])
                for j in range(d16):
                    rows_v[k, pl.ds(j * 16, 16)] = (
                        rows_v[k, pl.ds(j * 16, 16)] * pk)
                den_v[k, :] = pk
                return 0
            lax.fori_loop(0, K, scale, 0)

            pltpu.sync_copy(rows_v, accf_s.at[didx_v], add=True)
            pltpu.sync_copy(den_v, accd_s.at[didx_v], add=True)
            return 0
        lax.fori_loop(0, CH_PER_TILE, chunk, 0)
        plsc.subcore_barrier()

        for o, cnt in row_chunks:
            rsl = pl.ds(sid * ROWS_PER_TILE + o, cnt)
            pltpu.sync_copy(accf_s.at[rsl], accf_hbm.at[cid, rsl])
            pltpu.sync_copy(accd_s.at[rsl], accd_hbm.at[cid, rsl])

    return sc_edge


def kernel(x, edge_index, batch, W1, a1_src, a1_dst, b1, W2, a2_src, a2_dst,
           b2, W3, a3_src, a3_dst, b3, L1W, L1b, L2W, L2b):
    n, e = x.shape[0], edge_index.shape[1]
    x_pad = jnp.zeros((N_PAD, x.shape[1]), _f32).at[:n].set(x)
    pad_e = E_PAD - e
    src_p = jnp.concatenate(
        [edge_index[0], jnp.full((pad_e,), NN, jnp.int32)]).reshape(-1, K)
    dst_p = jnp.concatenate(
        [edge_index[1], jnp.full((pad_e,), NN, jnp.int32)]).reshape(-1, K)
    batch_p = jnp.concatenate(
        [batch, jnp.full((N_PAD - n,), NG, jnp.int32)]).reshape(1, N_PAD)

    h, astab, adtab, cvec = _tc_embed(x_pad, W1, a1_src, a1_dst)
    accf, accd = _make_sc_edge(W1.shape[1])(
        h, astab.reshape(-1), adtab.reshape(-1), cvec.reshape(-1),
        src_p, dst_p)
    h, astab, adtab, cvec = _tc_mid(accf, accd, b1, W2, a2_src, a2_dst)
    accf, accd = _make_sc_edge(W2.shape[1])(
        h, astab.reshape(-1), adtab.reshape(-1), cvec.reshape(-1),
        src_p, dst_p)
    h, astab, adtab, cvec = _tc_mid(accf, accd, b2, W3, a3_src, a3_dst)
    accf, accd = _make_sc_edge(W3.shape[1])(
        h, astab.reshape(-1), adtab.reshape(-1), cvec.reshape(-1),
        src_p, dst_p)
    return _tc_final(accf, accd, b3, batch_p, L1W, L1b, L2W, L2b)


# speedup vs baseline: 24.3007x; 1.3180x over previous
import functools

import jax
import jax.numpy as jnp
from jax import lax
from jax.experimental import pallas as pl
from jax.experimental.pallas import tpu as pltpu
from jax.experimental.pallas import tpu_sc as plsc

NN = 10000
NG = 64
N_PAD = 10176
NC, NS = 2, 16
NW = NC * NS
K = 128
CH_PER_TILE = 80
E_PAD = NW * CH_PER_TILE * K
N_CHUNK_ROWS = NW * CH_PER_TILE + 1
ROWS_PER_TILE = N_PAD // NS

_f32 = jnp.float32


def _tc_embed_body(hin_ref, w_ref, asrc_ref, adst_ref,
                   h_ref, astab_ref, adtab_ref, cvec_ref):
    h = jnp.dot(hin_ref[...], w_ref[...], preferred_element_type=_f32)
    h_ref[...] = h
    a_s = jnp.sum(h * asrc_ref[...], axis=1, keepdims=True)
    a_d = jnp.sum(h * adst_ref[...], axis=1, keepdims=True)
    astab_ref[...] = a_s
    adtab_ref[...] = a_d
    c = jnp.maximum(jnp.max(a_s) + jnp.max(a_d), 0.0)
    cvec_ref[...] = jnp.full((1, 16), c, _f32)


def _tc_embed(hin, w, asrc, adst):
    dout = w.shape[1]
    return pl.pallas_call(
        _tc_embed_body,
        out_shape=[
            jax.ShapeDtypeStruct((N_PAD, dout), _f32),
            jax.ShapeDtypeStruct((N_PAD, 1), _f32),
            jax.ShapeDtypeStruct((N_PAD, 1), _f32),
            jax.ShapeDtypeStruct((1, 16), _f32),
        ],
    )(hin, w, asrc.reshape(1, -1), adst.reshape(1, -1))


def _tc_mid_body(accf_ref, accd_ref, b_ref, w_ref, asrc_ref, adst_ref,
                 h_ref, astab_ref, adtab_ref, cvec_ref):
    feat = accf_ref[0] + accf_ref[1]
    den = accd_ref[0, :, 0:1] + accd_ref[1, :, 0:1]
    hprev = feat / (den + 1e-16) + b_ref[...]
    hprev = jnp.maximum(hprev, 0.0)
    h = jnp.dot(hprev, w_ref[...], preferred_element_type=_f32)
    h_ref[...] = h
    a_s = jnp.sum(h * asrc_ref[...], axis=1, keepdims=True)
    a_d = jnp.sum(h * adst_ref[...], axis=1, keepdims=True)
    astab_ref[...] = a_s
    adtab_ref[...] = a_d
    c = jnp.maximum(jnp.max(a_s) + jnp.max(a_d), 0.0)
    cvec_ref[...] = jnp.full((1, 16), c, _f32)


def _tc_mid(accf, accd, b, w, asrc, adst):
    dout = w.shape[1]
    return pl.pallas_call(
        _tc_mid_body,
        out_shape=[
            jax.ShapeDtypeStruct((N_PAD, dout), _f32),
            jax.ShapeDtypeStruct((N_PAD, 1), _f32),
            jax.ShapeDtypeStruct((N_PAD, 1), _f32),
            jax.ShapeDtypeStruct((1, 16), _f32),
        ],
    )(accf, accd, b.reshape(1, -1), w, asrc.reshape(1, -1), adst.reshape(1, -1))


def _tc_final_body(accf_ref, accd_ref, b_ref, batch_ref,
                   l1w_ref, l1b_ref, l2w_ref, l2b_ref, out_ref):
    feat = accf_ref[0] + accf_ref[1]
    den = accd_ref[0, :, 0:1] + accd_ref[1, :, 0:1]
    h = feat / (den + 1e-16) + b_ref[...]
    batch = batch_ref[...]
    gid = lax.broadcasted_iota(jnp.int32, (NG, N_PAD), 0)
    oh = (batch == gid).astype(_f32)
    g = jnp.dot(oh, h, preferred_element_type=_f32)
    g = jnp.maximum(jnp.dot(g, l1w_ref[...], preferred_element_type=_f32)
                    + l1b_ref[...], 0.0)
    z = jnp.dot(g, l2w_ref[...], preferred_element_type=_f32) + l2b_ref[...]
    m0 = jnp.max(z, axis=0, keepdims=True)
    z = z - m0
    out_ref[...] = z - jnp.log(jnp.sum(jnp.exp(z), axis=0, keepdims=True))


def _tc_final(accf, accd, b, batch_p, l1w, l1b, l2w, l2b):
    nclass = l2w.shape[1]
    return pl.pallas_call(
        _tc_final_body,
        out_shape=jax.ShapeDtypeStruct((NG, nclass), _f32),
    )(accf, accd, b.reshape(1, -1), batch_p,
      l1w, l1b.reshape(1, -1), l2w, l2b.reshape(1, -1))


@functools.cache
def _make_sc_edge(d):
    d16 = d // 16
    kk = 64 if d == 128 else K
    ncht = E_PAD // (NW * kk)
    mesh = plsc.VectorSubcoreMesh(core_axis_name="c", subcore_axis_name="s")

    @functools.partial(
        pl.kernel,
        out_type=(
            jax.ShapeDtypeStruct((NC, N_PAD, d), _f32),
            jax.ShapeDtypeStruct((NC, N_PAD, 16), _f32),
        ),
        mesh=mesh,
        compiler_params=pltpu.CompilerParams(
            needs_layout_passes=False, use_tc_tiling_on_sc=False),
        scratch_types=[
            pltpu.VMEM((N_PAD,), _f32),
            pltpu.VMEM((N_PAD,), _f32),
            pltpu.VMEM((16,), _f32),
            pltpu.VMEM((kk,), jnp.int32),
            pltpu.VMEM((kk,), jnp.int32),
            pltpu.VMEM((kk,), jnp.int32),
            pltpu.VMEM((kk,), jnp.int32),
            pltpu.VMEM((kk,), _f32),
            pltpu.VMEM((kk, d), _f32),
            pltpu.VMEM((kk, d), _f32),
            pltpu.VMEM((kk, 16), _f32),
            pltpu.VMEM_SHARED((N_PAD, d), _f32),
            pltpu.VMEM_SHARED((N_PAD, 16), _f32),
            pltpu.SemaphoreType.DMA,
            pltpu.SemaphoreType.DMA,
        ],
    )
    def sc_edge(h_hbm, astab_hbm, adtab_hbm, cvec_hbm, src_hbm, dst_hbm,
                accf_hbm, accd_hbm,
                as_v, ad_v, cv_v, sidx0, sidx1, didx0, didx1, p_v,
                rows0, rows1, den_v, accf_s, accd_s, semg0, semg1):
        sidx_v = [sidx0, sidx1]
        didx_v = [didx0, didx1]
        rows_v = [rows0, rows1]
        semg = [semg0, semg1]
        cid = lax.axis_index("c")
        sid = lax.axis_index("s")
        wid = sid * NC + cid

        def zrow(k, _):
            for j in range(d16):
                rows_v[0][k, pl.ds(j * 16, 16)] = jnp.zeros((16,), _f32)
            den_v[k, :] = jnp.zeros((16,), _f32)
            return 0
        lax.fori_loop(0, kk, zrow, 0)
        row_chunks = [(o, min(kk, ROWS_PER_TILE - o))
                      for o in range(0, ROWS_PER_TILE, kk)]
        for o, cnt in row_chunks:
            r0 = sid * ROWS_PER_TILE + o
            pltpu.sync_copy(rows_v[0].at[pl.ds(0, cnt)],
                            accf_s.at[pl.ds(r0, cnt)])
            pltpu.sync_copy(den_v.at[pl.ds(0, cnt)],
                            accd_s.at[pl.ds(r0, cnt)])

        pltpu.sync_copy(astab_hbm, as_v)
        pltpu.sync_copy(adtab_hbm, ad_v)
        pltpu.sync_copy(cvec_hbm, cv_v)
        plsc.subcore_barrier()

        def start_gather(ch, b):
            pltpu.sync_copy(src_hbm.at[ch], sidx_v[b])
            pltpu.sync_copy(dst_hbm.at[ch], didx_v[b])
            pltpu.async_copy(h_hbm.at[sidx_v[b]], rows_v[b], semg[b])

        def process(b):
            cv = cv_v[:]
            for g in range(kk // 16):
                s16 = sidx_v[b][pl.ds(g * 16, 16)]
                t16 = didx_v[b][pl.ds(g * 16, 16)]
                av = plsc.load_gather(as_v, [s16])
                bv = plsc.load_gather(ad_v, [t16])
                e = av + bv
                e = jnp.where(e >= 0.0, e, e * 0.2)
                p_v[pl.ds(g * 16, 16)] = jnp.exp(e - cv)
            pltpu.make_async_copy(h_hbm.at[sidx_v[b]], rows_v[b],
                                  semg[b]).wait()

            def scale(k4, _):
                for u in range(4):
                    k = k4 * 4 + u
                    pk = plsc.load_gather(p_v, [jnp.full((16,), k, jnp.int32)])
                    for j in range(d16):
                        rows_v[b][k, pl.ds(j * 16, 16)] = (
                            rows_v[b][k, pl.ds(j * 16, 16)] * pk)
                    den_v[k, :] = pk
                return 0
            lax.fori_loop(0, kk // 4, scale, 0)
            pltpu.sync_copy(rows_v[b], accf_s.at[didx_v[b]], add=True)
            pltpu.sync_copy(den_v, accd_s.at[didx_v[b]], add=True)

        base = wid * ncht
        start_gather(base, 0)

        def pipe(i, _):
            t0 = i * 2
            for b in range(2):
                start_gather(base + t0 + b + 1, 1 - b)
                process(b)
            return 0
        lax.fori_loop(0, ncht // 2, pipe, 0)
        pltpu.make_async_copy(h_hbm.at[sidx_v[0]], rows_v[0], semg[0]).wait()
        plsc.subcore_barrier()

        for o, cnt in row_chunks:
            rsl = pl.ds(sid * ROWS_PER_TILE + o, cnt)
            pltpu.sync_copy(accf_s.at[rsl], accf_hbm.at[cid, rsl])
            pltpu.sync_copy(accd_s.at[rsl], accd_hbm.at[cid, rsl])

    return sc_edge


def kernel(x, edge_index, batch, W1, a1_src, a1_dst, b1, W2, a2_src, a2_dst,
           b2, W3, a3_src, a3_dst, b3, L1W, L1b, L2W, L2b):
    n, e = x.shape[0], edge_index.shape[1]
    x_pad = jnp.zeros((N_PAD, x.shape[1]), _f32).at[:n].set(x)
    pad_e = E_PAD + K - e
    src_flat = jnp.concatenate(
        [edge_index[0], jnp.full((pad_e,), NN, jnp.int32)])
    dst_flat = jnp.concatenate(
        [edge_index[1], jnp.full((pad_e,), NN, jnp.int32)])

    def chunked(k):
        return (src_flat[:E_PAD + k].reshape(-1, k),
                dst_flat[:E_PAD + k].reshape(-1, k))
    batch_p = jnp.concatenate(
        [batch, jnp.full((N_PAD - n,), NG, jnp.int32)]).reshape(1, N_PAD)

    sp128, dp128 = chunked(128)
    sp64, dp64 = chunked(64)

    h, astab, adtab, cvec = _tc_embed(x_pad, W1, a1_src, a1_dst)
    accf, accd = _make_sc_edge(W1.shape[1])(
        h, astab.reshape(-1), adtab.reshape(-1), cvec.reshape(-1),
        sp128, dp128)
    h, astab, adtab, cvec = _tc_mid(accf, accd, b1, W2, a2_src, a2_dst)
    accf, accd = _make_sc_edge(W2.shape[1])(
        h, astab.reshape(-1), adtab.reshape(-1), cvec.reshape(-1),
        sp128, dp128)
    h, astab, adtab, cvec = _tc_mid(accf, accd, b2, W3, a3_src, a3_dst)
    accf, accd = _make_sc_edge(W3.shape[1])(
        h, astab.reshape(-1), adtab.reshape(-1), cvec.reshape(-1),
        sp64, dp64)
    return _tc_final(accf, accd, b3, batch_p, L1W, L1b, L2W, L2b)
